# initial kernel scaffold (unmeasured)
import jax
import jax.numpy as jnp
from jax import lax
from jax.experimental import pallas as pl
from jax.experimental.pallas import tpu as pltpu

N_DEV = 4


def kernel(x, w_mat):
    m_per, k = x.shape
    k2, n_per = w_mat.shape
    assert k == k2
    m_total = N_DEV * m_per

    def body(x_ref, w_ref, out_ref, comm_ref, max_buf, gather_buf,
             send_sems, recv_sems, s_send_sems, s_recv_sems, credit_sem):
        my = lax.axis_index("i")
        left = (my - 1) % N_DEV
        right = (my + 1) % N_DEV

        barrier_sem = pltpu.get_barrier_semaphore()
        for nbr in [left, right]:
            pl.semaphore_signal(
                barrier_sem, inc=1,
                device_id=(nbr,), device_id_type=pl.DeviceIdType.MESH,
            )
        pl.semaphore_wait(barrier_sem, 2)

        y0 = jnp.maximum(
            jnp.dot(x_ref[...], w_ref[...], preferred_element_type=jnp.float32),
            0.0,
        )
        out_ref[pl.ds(my * m_per, m_per), :] = y0

        srcs = [x_ref, comm_ref.at[1], comm_ref.at[0]]
        recv_slots = [1, 0, 1]
        send_idx = [0, 1, 0]
        for h in range(N_DEV - 1):
            if h == 2:
                pl.semaphore_wait(credit_sem, 1)
            rdma = pltpu.make_async_remote_copy(
                src_ref=srcs[h],
                dst_ref=comm_ref.at[recv_slots[h]],
                send_sem=send_sems.at[send_idx[h]],
                recv_sem=recv_sems.at[recv_slots[h]],
                device_id=(right,),
                device_id_type=pl.DeviceIdType.MESH,
            )
            rdma.start()
            rdma.wait()

            origin = (my - h - 1) % N_DEV
            chunk = comm_ref[recv_slots[h]]
            y = jnp.maximum(
                jnp.dot(chunk, w_ref[...], preferred_element_type=jnp.float32),
                0.0,
            )
            out_ref[pl.ds(origin * m_per, m_per), :] = y
            if h == 0:
                pl.semaphore_signal(
                    credit_sem, inc=1,
                    device_id=(left,), device_id_type=pl.DeviceIdType.MESH,
                )

        local_max = jnp.max(out_ref[...])
        max_buf[...] = jnp.full(max_buf.shape, local_max, jnp.float32)

        rdmas = []
        for d in (1, 2, 3):
            peer = (my + d) % N_DEV
            r = pltpu.make_async_remote_copy(
                src_ref=max_buf,
                dst_ref=gather_buf.at[my],
                send_sem=s_send_sems.at[d],
                recv_sem=s_recv_sems.at[d],
                device_id=(peer,),
                device_id_type=pl.DeviceIdType.MESH,
            )
            r.start()
            rdmas.append(r)
        gmax = local_max
        for d, r in zip((1, 2, 3), rdmas):
            r.wait()
            src_pos = (my - d) % N_DEV
            gmax = jnp.maximum(gmax, jnp.max(gather_buf[pl.ds(src_pos, 1)]))

        scale = gmax / 127.0
        q = jnp.clip(jnp.round(out_ref[...] / scale), -127.0, 127.0)
        out_ref[...] = q * scale

    return pl.pallas_call(
        body,
        out_shape=jax.ShapeDtypeStruct((m_total, n_per), jnp.float32),
        in_specs=[
            pl.BlockSpec(memory_space=pltpu.VMEM),
            pl.BlockSpec(memory_space=pltpu.VMEM),
        ],
        out_specs=pl.BlockSpec(memory_space=pltpu.VMEM),
        scratch_shapes=[
            pltpu.VMEM((2, m_per, k), jnp.float32),
            pltpu.VMEM((8, 128), jnp.float32),
            pltpu.VMEM((N_DEV, 8, 128), jnp.float32),
            pltpu.SemaphoreType.DMA((2,)),
            pltpu.SemaphoreType.DMA((2,)),
            pltpu.SemaphoreType.DMA((N_DEV,)),
            pltpu.SemaphoreType.DMA((N_DEV,)),
            pltpu.SemaphoreType.REGULAR,
        ],
        compiler_params=pltpu.CompilerParams(collective_id=0),
    )(x, w_mat)


# baseline (device time: 691848 ns/iter reference)
import jax
import jax.numpy as jnp
from jax import lax
from jax.experimental import pallas as pl
from jax.experimental.pallas import tpu as pltpu

N_DEV = 4


def _all_gather_x(x):
    m_per, k = x.shape

    def body(x_ref, xf_ref, copy_sem, send_sems, recv_sems):
        my = lax.axis_index("i")
        left = (my - 1) % N_DEV
        right = (my + 1) % N_DEV

        barrier_sem = pltpu.get_barrier_semaphore()
        for nbr in [left, right]:
            pl.semaphore_signal(
                barrier_sem, inc=1,
                device_id=(nbr,), device_id_type=pl.DeviceIdType.MESH,
            )
        pl.semaphore_wait(barrier_sem, 2)

        cp = pltpu.make_async_copy(x_ref, xf_ref.at[my], copy_sem)
        cp.start()
        cp.wait()

        for h in range(N_DEV - 1):
            origin = (my - h) % N_DEV
            rdma = pltpu.make_async_remote_copy(
                src_ref=xf_ref.at[origin],
                dst_ref=xf_ref.at[origin],
                send_sem=send_sems.at[h],
                recv_sem=recv_sems.at[h],
                device_id=(right,),
                device_id_type=pl.DeviceIdType.MESH,
            )
            rdma.start()
            rdma.wait()

    return pl.pallas_call(
        body,
        out_shape=jax.ShapeDtypeStruct((N_DEV, m_per, k), jnp.float32),
        in_specs=[pl.BlockSpec(memory_space=pl.ANY)],
        out_specs=pl.BlockSpec(memory_space=pl.ANY),
        scratch_shapes=[
            pltpu.SemaphoreType.DMA,
            pltpu.SemaphoreType.DMA((N_DEV - 1,)),
            pltpu.SemaphoreType.DMA((N_DEV - 1,)),
        ],
        compiler_params=pltpu.CompilerParams(collective_id=0),
    )(x)


def _gemm_relu_max(x_full, w):
    n_dev, m_per, k = x_full.shape
    _, n_per = w.shape
    bn = 512
    grid = (n_dev, n_per // bn)

    def body(x_ref, w_ref, y_ref, lmax_ref, acc_ref):
        m = pl.program_id(0)
        n = pl.program_id(1)

        @pl.when(jnp.logical_and(m == 0, n == 0))
        def _():
            acc_ref[...] = jnp.zeros_like(acc_ref)

        y = jnp.maximum(
            jnp.dot(x_ref[0], w_ref[...], preferred_element_type=jnp.float32),
            0.0,
        )
        y_ref[...] = y
        acc_ref[...] = jnp.maximum(acc_ref[...], jnp.max(y))
        lmax_ref[...] = acc_ref[...]

    return pl.pallas_call(
        body,
        grid=grid,
        out_shape=(
            jax.ShapeDtypeStruct((n_dev * m_per, n_per), jnp.float32),
            jax.ShapeDtypeStruct((8, 128), jnp.float32),
        ),
        in_specs=[
            pl.BlockSpec((1, m_per, k), lambda m, n: (m, 0, 0)),
            pl.BlockSpec((k, bn), lambda m, n: (0, n)),
        ],
        out_specs=(
            pl.BlockSpec((m_per, bn), lambda m, n: (m, n)),
            pl.BlockSpec((8, 128), lambda m, n: (0, 0)),
        ),
        scratch_shapes=[pltpu.VMEM((8, 128), jnp.float32)],
        compiler_params=pltpu.CompilerParams(
            vmem_limit_bytes=60 * 1024 * 1024
        ),
    )(x_full, w)


def _global_scale(lmax):

    def body(lmax_ref, scale_ref, gather_ref, send_sems, recv_sems):
        my = lax.axis_index("i")

        barrier_sem = pltpu.get_barrier_semaphore()
        for d in (1, 2, 3):
            pl.semaphore_signal(
                barrier_sem, inc=1,
                device_id=((my + d) % N_DEV,),
                device_id_type=pl.DeviceIdType.MESH,
            )
        pl.semaphore_wait(barrier_sem, 3)

        rdmas = []
        for d in (1, 2, 3):
            r = pltpu.make_async_remote_copy(
                src_ref=lmax_ref,
                dst_ref=gather_ref.at[my],
                send_sem=send_sems.at[d],
                recv_sem=recv_sems.at[d],
                device_id=((my + d) % N_DEV,),
                device_id_type=pl.DeviceIdType.MESH,
            )
            r.start()
            rdmas.append(r)
        gmax = jnp.max(lmax_ref[...])
        for d, r in zip((1, 2, 3), rdmas):
            r.wait()
            gmax = jnp.maximum(
                gmax, jnp.max(gather_ref[pl.ds((my - d) % N_DEV, 1)])
            )
        scale_ref[...] = jnp.full(scale_ref.shape, gmax / 127.0, jnp.float32)

    return pl.pallas_call(
        body,
        out_shape=jax.ShapeDtypeStruct((8, 128), jnp.float32),
        in_specs=[pl.BlockSpec(memory_space=pltpu.VMEM)],
        out_specs=pl.BlockSpec(memory_space=pltpu.VMEM),
        scratch_shapes=[
            pltpu.VMEM((N_DEV, 8, 128), jnp.float32),
            pltpu.SemaphoreType.DMA((N_DEV,)),
            pltpu.SemaphoreType.DMA((N_DEV,)),
        ],
        compiler_params=pltpu.CompilerParams(collective_id=1),
    )(lmax)


def _quantize(y, scale):
    m_total, n_per = y.shape
    bm = 512
    grid = (m_total // bm,)

    def body(y_ref, scale_ref, out_ref):
        s = scale_ref[0, 0]
        q = jnp.clip(jnp.round(y_ref[...] / s), -127.0, 127.0)
        out_ref[...] = q * s

    return pl.pallas_call(
        body,
        grid=grid,
        out_shape=jax.ShapeDtypeStruct((m_total, n_per), jnp.float32),
        in_specs=[
            pl.BlockSpec((bm, n_per), lambda i: (i, 0)),
            pl.BlockSpec((8, 128), lambda i: (0, 0)),
        ],
        out_specs=pl.BlockSpec((bm, n_per), lambda i: (i, 0)),
    )(y, scale)


def kernel(x, w_mat):
    x_full = _all_gather_x(x)
    y, lmax = _gemm_relu_max(x_full, w_mat)
    scale = _global_scale(lmax)
    return _quantize(y, scale)


# device time: 330962 ns/iter; 2.0904x vs baseline; 2.0904x over previous
import jax
import jax.numpy as jnp
from jax import lax
from jax.experimental import pallas as pl
from jax.experimental.pallas import tpu as pltpu

N_DEV = 4


def _ag_gemm_relu_max(x, w):
    m_per, k = x.shape
    _, n_per = w.shape
    half = m_per // 2
    P = 2
    pr = half // P
    NP = (N_DEV - 1) * P

    def body(x_ref, w_ref, y_ref, lmax_ref, xs_ref, gbuf, yt0, yt1,
             gbuf_sem, yt_sems, sendA, recvA, sendB, recvB):
        my = lax.axis_index("i")
        left = (my - 1) % N_DEV
        right = (my + 1) % N_DEV

        barrier_sem = pltpu.get_barrier_semaphore()
        for nbr in [left, right]:
            pl.semaphore_signal(
                barrier_sem, inc=1,
                device_id=(nbr,), device_id_type=pl.DeviceIdType.MESH,
            )
        pl.semaphore_wait(barrier_sem, 2)

        def gemm_piece(src_hbm, row_start, buf, sem, wait_first, mx):
            cp = pltpu.make_async_copy(src_hbm, gbuf, gbuf_sem)
            cp.start()
            cp.wait()
            ytile = jnp.maximum(
                jnp.dot(gbuf[...], w_ref[...],
                        preferred_element_type=jnp.float32),
                0.0,
            )
            d = pltpu.make_async_copy(
                buf, y_ref.at[pl.ds(row_start, pr), :], sem
            )
            if wait_first:
                d.wait()
            buf[...] = ytile
            d.start()
            return jnp.maximum(mx, jnp.max(ytile))

        def rdma_A(s, src, dst):
            return pltpu.make_async_remote_copy(
                src_ref=src, dst_ref=dst,
                send_sem=sendA.at[s], recv_sem=recvA.at[s],
                device_id=(right,), device_id_type=pl.DeviceIdType.MESH,
            )

        def rdma_B(s, src, dst):
            return pltpu.make_async_remote_copy(
                src_ref=src, dst_ref=dst,
                send_sem=sendB.at[s], recv_sem=recvB.at[s],
                device_id=(left,), device_id_type=pl.DeviceIdType.MESH,
            )

        for s in range(P):
            offA, offB = s * pr, half + s * pr
            rdma_A(s, x_ref.at[pl.ds(offA, pr), :],
                   xs_ref.at[my, pl.ds(offA, pr), :]).start()
            rdma_B(s, x_ref.at[pl.ds(offB, pr), :],
                   xs_ref.at[my, pl.ds(offB, pr), :]).start()

        def own_pair(i, mx, wait_first):
            r0 = my * m_per + (2 * i) * pr
            mx = gemm_piece(x_ref.at[pl.ds((2 * i) * pr, pr), :], r0,
                            yt0, yt_sems.at[0], wait_first, mx)
            mx = gemm_piece(x_ref.at[pl.ds((2 * i + 1) * pr, pr), :],
                            r0 + pr, yt1, yt_sems.at[1], wait_first, mx)
            return mx

        mx = own_pair(0, jnp.float32(0.0), False)
        mx = lax.fori_loop(1, P, lambda i, m: own_pair(i, m, True), mx)

        def hop(s, mx):
            offA = (s % P) * pr
            offB = half + offA
            oA = (my - 1 - s // P) % N_DEV
            slotA = xs_ref.at[oA, pl.ds(offA, pr), :]
            rdma_A(s, slotA, slotA).wait_recv()

            @pl.when(s + P < NP)
            def _():
                rdma_A(s + P, slotA, slotA).start()
            mx = gemm_piece(slotA, oA * m_per + offA,
                            yt0, yt_sems.at[0], True, mx)
            oB = (my + 1 + s // P) % N_DEV
            slotB = xs_ref.at[oB, pl.ds(offB, pr), :]
            rdma_B(s, slotB, slotB).wait_recv()

            @pl.when(s + P < NP)
            def _():
                rdma_B(s + P, slotB, slotB).start()
            mx = gemm_piece(slotB, oB * m_per + offB,
                            yt1, yt_sems.at[1], True, mx)
            return mx

        mx = lax.fori_loop(0, NP, hop, mx)

        def drain(s, c):
            dummy = xs_ref.at[0, pl.ds(0, pr), :]
            rdma_A(s, dummy, dummy).wait_send()
            rdma_B(s, dummy, dummy).wait_send()
            return c

        lax.fori_loop(0, NP, drain, 0)
        for buf, i in ((yt0, 0), (yt1, 1)):
            pltpu.make_async_copy(
                buf, y_ref.at[pl.ds(0, pr), :], yt_sems.at[i]
            ).wait()
        lmax_ref[...] = jnp.full(lmax_ref.shape, mx, jnp.float32)

    return pl.pallas_call(
        body,
        out_shape=(
            jax.ShapeDtypeStruct((N_DEV * m_per, n_per), jnp.float32),
            jax.ShapeDtypeStruct((8, 128), jnp.float32),
            jax.ShapeDtypeStruct((N_DEV, m_per, k), jnp.float32),
        ),
        in_specs=[
            pl.BlockSpec(memory_space=pl.ANY),
            pl.BlockSpec(memory_space=pltpu.VMEM),
        ],
        out_specs=(
            pl.BlockSpec(memory_space=pl.ANY),
            pl.BlockSpec(memory_space=pltpu.VMEM),
            pl.BlockSpec(memory_space=pl.ANY),
        ),
        scratch_shapes=[
            pltpu.VMEM((pr, k), jnp.float32),
            pltpu.VMEM((pr, n_per), jnp.float32),
            pltpu.VMEM((pr, n_per), jnp.float32),
            pltpu.SemaphoreType.DMA,
            pltpu.SemaphoreType.DMA((2,)),
            pltpu.SemaphoreType.DMA((NP,)),
            pltpu.SemaphoreType.DMA((NP,)),
            pltpu.SemaphoreType.DMA((NP,)),
            pltpu.SemaphoreType.DMA((NP,)),
        ],
        compiler_params=pltpu.CompilerParams(
            collective_id=0, vmem_limit_bytes=63 * 1024 * 1024
        ),
    )(x, w)


def _global_scale(lmax):

    def body(lmax_ref, scale_ref, gather_ref, send_sems, recv_sems):
        my = lax.axis_index("i")

        barrier_sem = pltpu.get_barrier_semaphore()
        for d in (1, 2, 3):
            pl.semaphore_signal(
                barrier_sem, inc=1,
                device_id=((my + d) % N_DEV,),
                device_id_type=pl.DeviceIdType.MESH,
            )
        pl.semaphore_wait(barrier_sem, 3)

        rdmas = []
        for d in (1, 2, 3):
            r = pltpu.make_async_remote_copy(
                src_ref=lmax_ref,
                dst_ref=gather_ref.at[my],
                send_sem=send_sems.at[d],
                recv_sem=recv_sems.at[d],
                device_id=((my + d) % N_DEV,),
                device_id_type=pl.DeviceIdType.MESH,
            )
            r.start()
            rdmas.append(r)
        gmax = jnp.max(lmax_ref[...])
        for d, r in zip((1, 2, 3), rdmas):
            r.wait()
            gmax = jnp.maximum(
                gmax, jnp.max(gather_ref[pl.ds((my - d) % N_DEV, 1)])
            )
        scale_ref[...] = jnp.full(scale_ref.shape, gmax / 127.0, jnp.float32)

    return pl.pallas_call(
        body,
        out_shape=jax.ShapeDtypeStruct((8, 128), jnp.float32),
        in_specs=[pl.BlockSpec(memory_space=pltpu.VMEM)],
        out_specs=pl.BlockSpec(memory_space=pltpu.VMEM),
        scratch_shapes=[
            pltpu.VMEM((N_DEV, 8, 128), jnp.float32),
            pltpu.SemaphoreType.DMA((N_DEV,)),
            pltpu.SemaphoreType.DMA((N_DEV,)),
        ],
        compiler_params=pltpu.CompilerParams(collective_id=1),
    )(lmax)


def _quantize(y, scale):
    m_total, n_per = y.shape
    bm = 512
    grid = (m_total // bm,)

    def body(y_ref, scale_ref, out_ref):
        s = scale_ref[0, 0]
        q = jnp.clip(jnp.round(y_ref[...] / s), -127.0, 127.0)
        out_ref[...] = q * s

    return pl.pallas_call(
        body,
        grid=grid,
        out_shape=jax.ShapeDtypeStruct((m_total, n_per), jnp.float32),
        in_specs=[
            pl.BlockSpec((bm, n_per), lambda i: (i, 0)),
            pl.BlockSpec((8, 128), lambda i: (0, 0)),
        ],
        out_specs=pl.BlockSpec((bm, n_per), lambda i: (i, 0)),
    )(y, scale)


def kernel(x, w_mat):
    y, lmax, _ = _ag_gemm_relu_max(x, w_mat)
    scale = _global_scale(lmax)
    return _quantize(y, scale)
